# baseline (speedup 1.0000x reference)
import functools

import jax
import jax.numpy as jnp
from jax import lax
from jax.experimental import pallas as pl
from jax.experimental.pallas import tpu as pltpu
from jax.experimental.pallas import tpu_sc as plsc

NUM_EMB = 1000000
D = 64
BATCH = 16384
HIST = 20
B = BATCH * HIST

NC, NS = 2, 16
NW = NC * NS
PER_W = B // NW
C = 128
CH = PER_W // C
NB = 4


def _make_gather():
  mesh = plsc.VectorSubcoreMesh(core_axis_name="c", subcore_axis_name="s")

  @functools.partial(
      pl.kernel,
      out_type=jax.ShapeDtypeStruct((B, D), jnp.float32),
      mesh=mesh,
      scratch_types=[
          pltpu.VMEM((CH, C), jnp.int32),
          pltpu.VMEM((NB, C, D), jnp.float32),
          pltpu.SemaphoreType.DMA((NB,)),
      ],
      compiler_params=pltpu.CompilerParams(use_tc_tiling_on_sc=False),
  )
  def gather_kernel(idx_hbm, table_hbm, out_hbm, idx_v, bufs, gsem):
    wid = lax.axis_index("s") * NC + lax.axis_index("c")
    base = wid * PER_W

    pltpu.sync_copy(idx_hbm.at[wid], idx_v)

    for b in range(NB):
      pltpu.async_copy(table_hbm.at[idx_v.at[b]], bufs.at[b], gsem.at[b])

    @pl.loop(0, CH - NB, step=NB)
    def _main(j0):
      for b in range(NB):
        j = j0 + b
        pltpu.make_async_copy(
            table_hbm.at[idx_v.at[j]], bufs.at[b], gsem.at[b]
        ).wait()
        pltpu.sync_copy(bufs.at[b], out_hbm.at[pl.ds(base + j * C, C)])
        pltpu.async_copy(
            table_hbm.at[idx_v.at[j + NB]], bufs.at[b], gsem.at[b]
        )

    for b in range(NB):
      j = CH - NB + b
      pltpu.make_async_copy(
          table_hbm.at[idx_v.at[j]], bufs.at[b], gsem.at[b]
      ).wait()
      pltpu.sync_copy(bufs.at[b], out_hbm.at[pl.ds(base + j * C, C)])

  return gather_kernel


_gather = _make_gather()


@jax.jit
def kernel(input, table):
  idx = input.reshape(NW, CH, C)
  out = _gather(idx, table)
  return out.reshape(BATCH, HIST, D)
